# baseline (speedup 1.0000x reference)
import functools

import jax
import jax.numpy as jnp
from jax.experimental import pallas as pl
from jax.experimental.pallas import tpu as pltpu

B = 4
E = 8
C = 512
T = B * 16 * 16
EPAD = 128
BLK = 128
NB = T // BLK + E
PAD = NB * BLK


def _gate_body(tok_ref, gw1_ref, gb1_ref, gw2_ref, gb2_ref, idx_ref, p_ref):
    g1 = jax.nn.gelu(
        jax.lax.dot_general(tok_ref[...], gw1_ref[...],
                            (((1,), (0,)), ((), ())),
                            preferred_element_type=jnp.float32)
        + gb1_ref[...])
    logits = jax.lax.dot_general(g1, gw2_ref[...],
                                 (((1,), (0,)), ((), ())),
                                 preferred_element_type=jnp.float32)
    logits = logits + gb2_ref[...]
    col = jax.lax.broadcasted_iota(jnp.int32, (T, EPAD), 1)
    logits = jnp.where(col < E, logits, -1e30)
    m = jnp.max(logits, axis=1, keepdims=True)
    ex = jnp.exp(logits - m)
    denom = jnp.sum(ex, axis=1, keepdims=True)
    p_ref[...] = 1.0 / denom
    idx_ref[...] = jnp.min(jnp.where(logits == m, col, EPAD),
                           axis=1, keepdims=True)


def _gate(tok, gate_w1, gate_b1, gate_w2, gate_b2):
    gw2p = jnp.zeros((C, EPAD), jnp.float32).at[:, :E].set(gate_w2)
    gb2p = jnp.zeros((1, EPAD), jnp.float32).at[0, :E].set(gate_b2)
    return pl.pallas_call(
        _gate_body,
        out_shape=(jax.ShapeDtypeStruct((T, 1), jnp.int32),
                   jax.ShapeDtypeStruct((T, 1), jnp.float32)),
    )(tok, gate_w1, gate_b1.reshape(1, C), gw2p, gb2p)


def _experts_body(bs_ref, x_ref, w1_ref, b1_ref, w2_ref, b2_ref, out_ref):
    j = pl.program_id(0)

    @pl.when(j < bs_ref[E])
    def _():
        h = jax.nn.gelu(
            jax.lax.dot_general(x_ref[...], w1_ref[0],
                                (((1,), (0,)), ((), ())),
                                preferred_element_type=jnp.float32)
            + b1_ref[0])
        out_ref[...] = jax.lax.dot_general(
            h, w2_ref[0], (((1,), (0,)), ((), ())),
            preferred_element_type=jnp.float32) + b2_ref[0]


def _expert_of_block(j, bs_ref):
    e = jnp.int32(0)
    for k in range(1, E):
        e = e + jnp.where(j >= bs_ref[k], 1, 0).astype(jnp.int32)
    return e


def _grouped_experts(x_padded, bs, exp_w1, exp_b1, exp_w2, exp_b2):
    grid_spec = pltpu.PrefetchScalarGridSpec(
        num_scalar_prefetch=1,
        grid=(NB,),
        in_specs=[
            pl.BlockSpec((BLK, C), lambda j, bs_ref: (j, 0)),
            pl.BlockSpec((1, C, C),
                         lambda j, bs_ref: (_expert_of_block(j, bs_ref), 0, 0)),
            pl.BlockSpec((1, 1, C),
                         lambda j, bs_ref: (_expert_of_block(j, bs_ref), 0, 0)),
            pl.BlockSpec((1, C, C),
                         lambda j, bs_ref: (_expert_of_block(j, bs_ref), 0, 0)),
            pl.BlockSpec((1, 1, C),
                         lambda j, bs_ref: (_expert_of_block(j, bs_ref), 0, 0)),
        ],
        out_specs=pl.BlockSpec((BLK, C), lambda j, bs_ref: (j, 0)),
    )
    return pl.pallas_call(
        _experts_body,
        grid_spec=grid_spec,
        out_shape=jax.ShapeDtypeStruct((PAD, C), jnp.float32),
    )(bs, x_padded, exp_w1, exp_b1.reshape(E, 1, C),
      exp_w2, exp_b2.reshape(E, 1, C))


def kernel(s4, s8, s16, s32, gate_w1, gate_b1, gate_w2, gate_b2,
           exp_w1, exp_b1, exp_w2, exp_b2):
    tok = jnp.transpose(s32, (0, 2, 3, 1)).reshape(T, C)

    idx2, p2 = _gate(tok, gate_w1, gate_b1, gate_w2, gate_b2)
    idx = idx2[:, 0]

    order = jnp.argsort(idx, stable=True)
    counts = jnp.zeros((E,), jnp.int32).at[idx].add(1)
    nblk = (counts + BLK - 1) // BLK
    bs = jnp.concatenate([jnp.zeros((1,), jnp.int32),
                          jnp.cumsum(nblk)]).astype(jnp.int32)
    compact_off = jnp.concatenate([jnp.zeros((1,), jnp.int32),
                                   jnp.cumsum(counts)[:-1]]).astype(jnp.int32)
    sorted_e = idx[order]
    rank = jnp.arange(T, dtype=jnp.int32) - compact_off[sorted_e]
    dest_sorted = bs[sorted_e] * BLK + rank
    dest = jnp.zeros((T,), jnp.int32).at[order].set(dest_sorted)
    x_padded = jnp.zeros((PAD, C), jnp.float32).at[dest].set(tok)

    bs16 = jnp.full((16,), NB, jnp.int32).at[:E + 1].set(bs)
    y_padded = _grouped_experts(x_padded, bs16, exp_w1, exp_b1, exp_w2, exp_b2)

    y_tok = y_padded[dest] * p2 + tok

    s32_out = jnp.transpose(y_tok.reshape(B, 16, 16, C), (0, 3, 1, 2))
    return (s4, s8, s16, s32_out)


# speedup vs baseline: 1.4378x; 1.4378x over previous
import jax
import jax.numpy as jnp
from jax import lax
from jax.experimental import pallas as pl
from jax.experimental.pallas import tpu as pltpu
from jax.experimental.pallas import tpu_sc as plsc

B = 4
E = 8
C = 512
T = B * 16 * 16
EPAD = 128
BLK = 128
NB = T // BLK + E
PAD = NB * BLK
NC, NS, L = 2, 16, 16
NW = NC * NS
CH = T // NW
NV = T // L


def _gate_body(tok_ref, gw1_ref, gb1_ref, gw2_ref, gb2_ref, idx_ref, p_ref):
    g1 = jax.nn.gelu(
        jax.lax.dot_general(tok_ref[...], gw1_ref[...],
                            (((1,), (0,)), ((), ())),
                            preferred_element_type=jnp.float32)
        + gb1_ref[...])
    logits = jax.lax.dot_general(g1, gw2_ref[...],
                                 (((1,), (0,)), ((), ())),
                                 preferred_element_type=jnp.float32)
    logits = logits + gb2_ref[...]
    col = jax.lax.broadcasted_iota(jnp.int32, (T, EPAD), 1)
    logits = jnp.where(col < E, logits, -1e30)
    m = jnp.max(logits, axis=1, keepdims=True)
    ex = jnp.exp(logits - m)
    denom = jnp.sum(ex, axis=1, keepdims=True)
    p_ref[...] = jnp.broadcast_to(1.0 / denom, (T, EPAD))
    idx_ref[...] = jnp.min(jnp.where(logits == m, col, EPAD),
                           axis=1, keepdims=True)


def _gate(tok, gate_w1, gate_b1, gate_w2, gate_b2):
    gw2p = jnp.zeros((C, EPAD), jnp.float32).at[:, :E].set(gate_w2)
    gb2p = jnp.zeros((1, EPAD), jnp.float32).at[0, :E].set(gate_b2)
    return pl.pallas_call(
        _gate_body,
        out_shape=(jax.ShapeDtypeStruct((T, 1), jnp.int32),
                   jax.ShapeDtypeStruct((T, EPAD), jnp.float32)),
    )(tok, gate_w1, gate_b1.reshape(1, C), gw2p, gb2p)


def _vgather(v, i):
    return lax.gather(
        v, i[:, None],
        lax.GatherDimensionNumbers(offset_dims=(), collapsed_slice_dims=(0,),
                                   start_index_map=(0,)),
        slice_sizes=(1,),
        mode=lax.GatherScatterMode.PROMISE_IN_BOUNDS)


def _worker_dispatch_math(read_vreg, wid):
    iota = lax.iota(jnp.int32, L)
    zero = jnp.zeros((L,), jnp.int32)

    def hist_step(k, carry):
        tot, pre = carry
        v = read_vreg(k)
        flag = jnp.where(k < 2 * wid, 1, 0)
        new_tot, new_pre = [], []
        for e in range(E):
            m = jnp.where(v == e, 1, 0)
            new_tot.append(tot[e] + m)
            new_pre.append(pre[e] + m * flag)
        return tuple(new_tot), tuple(new_pre)

    tot, pre = lax.fori_loop(0, NV, hist_step,
                             (tuple(zero for _ in range(E)),
                              tuple(zero for _ in range(E))))
    hv = zero
    pv = zero
    for e in range(E):
        lane = jnp.where(iota == e, 1, 0)
        hv = hv + lane * jnp.sum(tot[e])
        pv = pv + lane * jnp.sum(pre[e])

    nblk = (hv + (BLK - 1)) // BLK
    bs_incl = jnp.cumsum(nblk)
    padded_off = (bs_incl - nblk) * BLK
    base = padded_off + pv

    a = read_vreg(2 * wid)
    b = read_vreg(2 * wid + 1)
    cnt_a = zero
    intra_a = zero
    intra_b = zero
    for e in range(E):
        ma = jnp.where(a == e, 1, 0)
        mb = jnp.where(b == e, 1, 0)
        ca = jnp.cumsum(ma)
        cb = jnp.cumsum(mb)
        intra_a = jnp.where(a == e, ca - 1, intra_a)
        cnt_a = cnt_a + jnp.where(iota == e, 1, 0) * jnp.sum(ma)
        intra_b = jnp.where(b == e, cb - 1, intra_b)
    dest_a = _vgather(base, a) + intra_a
    dest_b = _vgather(base + cnt_a, b) + intra_b

    shifted = _vgather(bs_incl, jnp.maximum(iota - 1, 0))
    bsx = shifted * jnp.where(iota == 0, 0, 1)
    bsx = bsx * jnp.where(iota > E, 0, 1) + jnp.where(iota > E, NB, 0)
    return dest_a, dest_b, bsx


def _dispatch_body(idx_hbm, tok_hbm, prep_hbm, xpad_hbm, ppad_hbm, dest_hbm,
                   bs_hbm, idx_all, dest_v, rows_v, prow_v, bs_v, sem):
    wid = lax.axis_index("s") * NC + lax.axis_index("c")
    pltpu.sync_copy(idx_hbm, idx_all)
    read = lambda k: idx_all[pl.ds(k * L, L)]
    dest_a, dest_b, bsx = _worker_dispatch_math(read, wid)

    dest_v[pl.ds(0, L)] = dest_a
    dest_v[pl.ds(L, L)] = dest_b
    pltpu.sync_copy(dest_v, dest_hbm.at[pl.ds(wid * CH, CH)])

    pltpu.sync_copy(tok_hbm.at[pl.ds(wid * CH, CH)], rows_v)
    pltpu.sync_copy(prep_hbm.at[pl.ds(wid * CH, CH)], prow_v)
    c1 = pltpu.async_copy(rows_v, xpad_hbm.at[dest_v], sem)
    c2 = pltpu.async_copy(prow_v, ppad_hbm.at[dest_v], sem)
    c1.wait()
    c2.wait()

    @pl.when(wid == 0)
    def _():
        bs_v[...] = bsx
        pltpu.sync_copy(bs_v, bs_hbm)


def _dispatch(idx, tok, prep):
    mesh = plsc.VectorSubcoreMesh(core_axis_name="c", subcore_axis_name="s",
                                  num_cores=NC, num_subcores=NS)
    return pl.kernel(
        _dispatch_body,
        out_type=(jax.ShapeDtypeStruct((PAD, C), jnp.float32),
                  jax.ShapeDtypeStruct((PAD, EPAD), jnp.float32),
                  jax.ShapeDtypeStruct((T,), jnp.int32),
                  jax.ShapeDtypeStruct((16,), jnp.int32)),
        mesh=mesh,
        scratch_types=[
            pltpu.VMEM((T,), jnp.int32),
            pltpu.VMEM((CH,), jnp.int32),
            pltpu.VMEM((CH, C), jnp.float32),
            pltpu.VMEM((CH, EPAD), jnp.float32),
            pltpu.VMEM((L,), jnp.int32),
            pltpu.SemaphoreType.DMA,
        ],
        compiler_params=pltpu.CompilerParams(needs_layout_passes=False),
    )(idx, tok, prep)


def _experts_body(bs_ref, x_ref, p_ref, w1_ref, b1_ref, w2_ref, b2_ref,
                  out_ref):
    j = pl.program_id(0)

    @pl.when(j < bs_ref[E])
    def _():
        h = jax.nn.gelu(
            jax.lax.dot_general(x_ref[...], w1_ref[0],
                                (((1,), (0,)), ((), ())),
                                preferred_element_type=jnp.float32)
            + b1_ref[0])
        y = jax.lax.dot_general(
            h, w2_ref[0], (((1,), (0,)), ((), ())),
            preferred_element_type=jnp.float32) + b2_ref[0]
        out_ref[...] = y * p_ref[:, :1] + x_ref[...]


def _expert_of_block(j, bs_ref):
    e = jnp.int32(0)
    for k in range(1, E):
        e = e + jnp.where(j >= bs_ref[k], 1, 0).astype(jnp.int32)
    return e


def _grouped_experts(x_padded, p_padded, bs, exp_w1, exp_b1, exp_w2, exp_b2):
    grid_spec = pltpu.PrefetchScalarGridSpec(
        num_scalar_prefetch=1,
        grid=(NB,),
        in_specs=[
            pl.BlockSpec((BLK, C), lambda j, bs_ref: (j, 0)),
            pl.BlockSpec((BLK, EPAD), lambda j, bs_ref: (j, 0)),
            pl.BlockSpec((1, C, C),
                         lambda j, bs_ref: (_expert_of_block(j, bs_ref), 0, 0)),
            pl.BlockSpec((1, 1, C),
                         lambda j, bs_ref: (_expert_of_block(j, bs_ref), 0, 0)),
            pl.BlockSpec((1, C, C),
                         lambda j, bs_ref: (_expert_of_block(j, bs_ref), 0, 0)),
            pl.BlockSpec((1, 1, C),
                         lambda j, bs_ref: (_expert_of_block(j, bs_ref), 0, 0)),
        ],
        out_specs=pl.BlockSpec((BLK, C), lambda j, bs_ref: (j, 0)),
    )
    return pl.pallas_call(
        _experts_body,
        grid_spec=grid_spec,
        out_shape=jax.ShapeDtypeStruct((PAD, C), jnp.float32),
    )(bs, x_padded, p_padded, exp_w1, exp_b1.reshape(E, 1, C),
      exp_w2, exp_b2.reshape(E, 1, C))


def _combine_body(ypad_hbm, dest_hbm, out_hbm, dest_v, rows_v, sem):
    wid = lax.axis_index("s") * NC + lax.axis_index("c")
    base = wid * CH
    pltpu.sync_copy(dest_hbm.at[pl.ds(base, CH)], dest_v)
    pltpu.async_copy(ypad_hbm.at[dest_v], rows_v, sem).wait()
    pltpu.sync_copy(rows_v, out_hbm.at[pl.ds(base, CH)])


def _combine(y_padded, dest):
    mesh = plsc.VectorSubcoreMesh(core_axis_name="c", subcore_axis_name="s",
                                  num_cores=NC, num_subcores=NS)
    return pl.kernel(
        _combine_body,
        out_type=jax.ShapeDtypeStruct((T, C), jnp.float32),
        mesh=mesh,
        scratch_types=[
            pltpu.VMEM((CH,), jnp.int32),
            pltpu.VMEM((CH, C), jnp.float32),
            pltpu.SemaphoreType.DMA,
        ],
        compiler_params=pltpu.CompilerParams(needs_layout_passes=False),
    )(y_padded, dest)


def kernel(s4, s8, s16, s32, gate_w1, gate_b1, gate_w2, gate_b2,
           exp_w1, exp_b1, exp_w2, exp_b2):
    tok = jnp.transpose(s32, (0, 2, 3, 1)).reshape(T, C)

    idx2, prep = _gate(tok, gate_w1, gate_b1, gate_w2, gate_b2)
    x_padded, p_padded, dest, bs = _dispatch(idx2.reshape(T), tok, prep)
    y_padded = _grouped_experts(x_padded, p_padded, bs,
                                exp_w1, exp_b1, exp_w2, exp_b2)
    y_tok = _combine(y_padded, dest)

    s32_out = jnp.transpose(y_tok.reshape(B, 16, 16, C), (0, 3, 1, 2))
    return (s4, s8, s16, s32_out)
